# baseline (speedup 1.0000x reference)
import functools

import jax
import jax.numpy as jnp
from jax import lax
from jax.experimental import pallas as pl
from jax.experimental.pallas import tpu as pltpu
from jax.experimental.pallas import tpu_sc as plsc

VOCAB = 100000
D_MODEL = 768
MAX_LEN = 2048
BATCH = 4

_NC = 2
_NS = 16
_NW = _NC * _NS

_CHUNK = 128


def _sc_gather(table, flat_ids, offset, b):
    b_per_w = b // _NW
    n_chunks = b_per_w // _CHUNK
    mesh = plsc.VectorSubcoreMesh(core_axis_name="c", subcore_axis_name="s")

    n_bufs = min(2, n_chunks)
    scratch = [pltpu.VMEM((b_per_w,), jnp.int32)]
    scratch += [pltpu.VMEM((_CHUNK, D_MODEL), jnp.float32)] * n_bufs
    scratch += [pltpu.SemaphoreType.DMA] * (2 * n_bufs)

    @functools.partial(
        pl.kernel,
        mesh=mesh,
        out_type=jax.ShapeDtypeStruct((b, D_MODEL), jnp.float32),
        scratch_types=scratch,
    )
    def gather_kernel(table_hbm, idx_hbm, out_hbm, idx_v, *rest):
        bufs = rest[:n_bufs]
        gsems = rest[n_bufs:2 * n_bufs]
        wsems = rest[2 * n_bufs:]
        wid = lax.axis_index("s") * _NC + lax.axis_index("c")
        base = wid * b_per_w
        pltpu.sync_copy(idx_hbm.at[pl.ds(offset + base, b_per_w)], idx_v)

        g_copies = [None] * n_chunks
        w_copies = [None] * n_chunks

        def start_gather(c):
            g_copies[c] = pltpu.async_copy(
                table_hbm.at[idx_v.at[pl.ds(c * _CHUNK, _CHUNK)]],
                bufs[c % n_bufs], gsems[c % n_bufs])

        def start_write(c):
            w_copies[c] = pltpu.async_copy(
                bufs[c % n_bufs],
                out_hbm.at[pl.ds(base + c * _CHUNK, _CHUNK)],
                wsems[c % n_bufs])

        start_gather(0)
        if n_chunks > 1:
            start_gather(1)
        for c in range(n_chunks):
            g_copies[c].wait()
            start_write(c)
            nxt = c + n_bufs
            if nxt < n_chunks:
                w_copies[c].wait()
                start_gather(nxt)
        for c in range(max(0, n_chunks - n_bufs), n_chunks):
            w_copies[c].wait()

    return gather_kernel(table, flat_ids)


def _ln_body(x_ref, pos_ref, gamma_ref, beta_ref, *rest):
    o_ref = rest[-1]
    x = x_ref[...] + pos_ref[...]
    mean = jnp.mean(x, axis=-1, keepdims=True)
    xc = x - mean
    var = jnp.mean(xc * xc, axis=-1, keepdims=True)
    o_ref[0] = xc * lax.rsqrt(var + 1e-5) * gamma_ref[...] + beta_ref[...]


def _tc_add_ln_chunk(gathered, pos_emb, gamma, beta, total_batch,
                     batch_off, prev):
    nb = gathered.shape[0] // MAX_LEN
    in_specs = [
        pl.BlockSpec((MAX_LEN, D_MODEL), lambda b: (b, 0)),
        pl.BlockSpec((MAX_LEN, D_MODEL), lambda b: (0, 0)),
        pl.BlockSpec((1, D_MODEL), lambda b: (0, 0)),
        pl.BlockSpec((1, D_MODEL), lambda b: (0, 0)),
    ]
    args = [gathered, pos_emb, gamma, beta]
    aliases = {}
    if prev is not None:
        in_specs.append(pl.BlockSpec(memory_space=pl.ANY))
        args.append(prev)
        aliases = {4: 0}
    return pl.pallas_call(
        _ln_body,
        grid=(nb,),
        in_specs=in_specs,
        out_specs=pl.BlockSpec((1, MAX_LEN, D_MODEL),
                               lambda b: (b + batch_off, 0, 0)),
        out_shape=jax.ShapeDtypeStruct((total_batch, MAX_LEN, D_MODEL),
                                       jnp.float32),
        input_output_aliases=aliases,
    )(*args)


_N_CHUNKS = 2


@jax.jit
def kernel(token_ids, word_table, pos_emb, gamma, beta):
    n_batch = token_ids.shape[0]
    step = n_batch // _N_CHUNKS
    flat_ids = token_ids.reshape(-1).astype(jnp.int32)
    gamma2 = gamma.reshape(1, D_MODEL)
    beta2 = beta.reshape(1, D_MODEL)
    gathered = [
        _sc_gather(word_table,
                   flat_ids[k * step * MAX_LEN:(k + 1) * step * MAX_LEN],
                   0, step * MAX_LEN)
        for k in range(_N_CHUNKS)
    ]
    buf = None
    for k in range(_N_CHUNKS):
        buf = _tc_add_ln_chunk(gathered[k], pos_emb, gamma2, beta2,
                               n_batch, k * step, buf)
    return buf
